# Initial kernel scaffold; baseline (speedup 1.0000x reference)
import jax
import jax.numpy as jnp
from jax.experimental import pallas as pl


def kernel(node_attr, edge_attr, edge_index, W1, b1, W2, b2, Wn1, bn1, Wn2, bn2, Wn3, bn3, We1, be1, We2, be2, We3, be3):
    raise NotImplementedError("write your pallas kernel here")


# speedup vs baseline: 4.0822x; 4.0822x over previous
import functools

import jax
import jax.numpy as jnp
from jax import lax
from jax.experimental import pallas as pl
from jax.experimental.pallas import tpu as pltpu
from jax.experimental.pallas import tpu_sc as plsc

N = 10000
E = 320000
NP = 10240
GC = 128

NCORE = 2
NSUB = 16
NW = NCORE * NSUB
EPW = E // NW
CH = 80
NCHUNK = EPW // CH

BE = 1280
BN = 1280


def _leaky(x):
    return jnp.where(x > 0, x, 0.2 * x)


def _edge_block_kernel(ea, w1e, b1, w2, b2, we1, be1, we2, be2, we3, be3,
                       eo_ref, ef_ref):
    h = jnp.maximum(
        jnp.dot(ea[...], w1e[...], preferred_element_type=jnp.float32)
        + b1[...], 0.0)
    eo = jnp.dot(h, w2[...], preferred_element_type=jnp.float32) + b2[...]
    eo_ref[...] = eo
    e1 = _leaky(eo)
    h1 = _leaky(jnp.dot(e1, we1[...], preferred_element_type=jnp.float32)
                + be1[...])
    h2 = _leaky(jnp.dot(h1, we2[...], preferred_element_type=jnp.float32)
                + be2[...])
    ef_ref[...] = (jnp.dot(h2, we3[...], preferred_element_type=jnp.float32)
                   + be3[...])


def _node_block_kernel(sums, cnts, wn1, bn1, wn2, bn2, wn3, bn3, out_ref):
    s = sums[0] + sums[1]
    c = cnts[0, :, 0:1] + cnts[1, :, 0:1]
    nod = s / jnp.maximum(c, 1.0)
    n1 = _leaky(nod)
    h1 = _leaky(jnp.dot(n1, wn1[...], preferred_element_type=jnp.float32)
                + bn1[...])
    h2 = _leaky(jnp.dot(h1, wn2[...], preferred_element_type=jnp.float32)
                + bn2[...])
    out_ref[...] = (jnp.dot(h2, wn3[...], preferred_element_type=jnp.float32)
                    + bn3[...])


def _scatter_sums_body(eo_hbm, row_hbm, sums_out, idx_v, rows_v, zrow_v,
                       sum_sh):
    c = lax.axis_index("c")
    s = lax.axis_index("s")
    wid = s * NCORE + c

    zeros16 = jnp.zeros((16,), jnp.float32)
    for i in range(16):
        for j in range(GC // 16):
            zrow_v[i, 16 * j:16 * (j + 1)] = zeros16

    stripe = NP // NSUB
    nb = stripe // 16

    def _zero_blk(i, carry):
        base = s * stripe + i * 16
        pltpu.sync_copy(zrow_v, sum_sh.at[pl.ds(base, 16)])
        return carry

    lax.fori_loop(0, nb, _zero_blk, 0)
    plsc.subcore_barrier()

    ebase = wid * EPW

    def _chunk(k, carry):
        base = ebase + k * CH
        pltpu.sync_copy(row_hbm.at[pl.ds(base, CH)], idx_v)
        pltpu.sync_copy(eo_hbm.at[pl.ds(base, CH)], rows_v)
        pltpu.sync_copy(rows_v, sum_sh.at[idx_v], add=True)
        return carry

    lax.fori_loop(0, NCHUNK, _chunk, 0)
    plsc.subcore_barrier()

    def _out_blk(i, carry):
        base = s * stripe + i * 16
        pltpu.sync_copy(sum_sh.at[pl.ds(base, 16)],
                        sums_out.at[c, pl.ds(base, 16)])
        return carry

    lax.fori_loop(0, nb, _out_blk, 0)


def _scatter_cnts_body(row_hbm, cnts_out, idx_v, ones_v, zrow_v, cnt_sh):
    c = lax.axis_index("c")
    s = lax.axis_index("s")
    wid = s * NCORE + c

    zeros16 = jnp.zeros((16,), jnp.float32)
    ones16 = jnp.ones((16,), jnp.float32)
    for i in range(16):
        for j in range(GC // 16):
            zrow_v[i, 16 * j:16 * (j + 1)] = zeros16

    def _init_ones(i, carry):
        for j in range(GC // 16):
            ones_v[i, 16 * j:16 * (j + 1)] = ones16
        return carry

    lax.fori_loop(0, CH, _init_ones, 0)

    stripe = NP // NSUB
    nb = stripe // 16

    def _zero_blk(i, carry):
        base = s * stripe + i * 16
        pltpu.sync_copy(zrow_v, cnt_sh.at[pl.ds(base, 16)])
        return carry

    lax.fori_loop(0, nb, _zero_blk, 0)
    plsc.subcore_barrier()

    ebase = wid * EPW

    def _chunk(k, carry):
        base = ebase + k * CH
        pltpu.sync_copy(row_hbm.at[pl.ds(base, CH)], idx_v)
        pltpu.sync_copy(ones_v, cnt_sh.at[idx_v], add=True)
        return carry

    lax.fori_loop(0, NCHUNK, _chunk, 0)
    plsc.subcore_barrier()

    def _out_blk(i, carry):
        base = s * stripe + i * 16
        pltpu.sync_copy(cnt_sh.at[pl.ds(base, 16)],
                        cnts_out.at[c, pl.ds(base, 16)])
        return carry

    lax.fori_loop(0, nb, _out_blk, 0)


_MESH = plsc.VectorSubcoreMesh(core_axis_name="c", subcore_axis_name="s",
                               num_cores=NCORE, num_subcores=NSUB)

_scatter_sums = functools.partial(
    pl.kernel,
    out_type=jax.ShapeDtypeStruct((NCORE, NP, GC), jnp.float32),
    mesh=_MESH,
    scratch_types=[
        pltpu.VMEM((CH,), jnp.int32),
        pltpu.VMEM((CH, GC), jnp.float32),
        pltpu.VMEM((16, GC), jnp.float32),
        pltpu.VMEM_SHARED((NP, GC), jnp.float32),
    ],
)(_scatter_sums_body)

_scatter_cnts = functools.partial(
    pl.kernel,
    out_type=jax.ShapeDtypeStruct((NCORE, NP, GC), jnp.float32),
    mesh=_MESH,
    scratch_types=[
        pltpu.VMEM((CH,), jnp.int32),
        pltpu.VMEM((CH, GC), jnp.float32),
        pltpu.VMEM((16, GC), jnp.float32),
        pltpu.VMEM_SHARED((NP, GC), jnp.float32),
    ],
)(_scatter_cnts_body)


def kernel(node_attr, edge_attr, edge_index, W1, b1, W2, b2,
           Wn1, bn1, Wn2, bn2, Wn3, bn3,
           We1, be1, We2, be2, We3, be3):
    nc = node_attr.shape[1]
    w1e = W1[2 * nc:]
    b1r = b1.reshape(1, -1)
    b2r = b2.reshape(1, -1)
    bn1r, bn2r, bn3r = bn1.reshape(1, -1), bn2.reshape(1, -1), bn3.reshape(1, -1)
    be1r, be2r, be3r = be1.reshape(1, -1), be2.reshape(1, -1), be3.reshape(1, -1)
    row = edge_index[0]

    full = lambda shape: pl.BlockSpec(shape, lambda i: (0,) * len(shape))
    eo, edge_final = pl.pallas_call(
        _edge_block_kernel,
        grid=(E // BE,),
        in_specs=[
            pl.BlockSpec((BE, edge_attr.shape[1]), lambda i: (i, 0)),
            full(w1e.shape), full(b1r.shape),
            full(W2.shape), full(b2r.shape),
            full(We1.shape), full(be1r.shape),
            full(We2.shape), full(be2r.shape),
            full(We3.shape), full(be3r.shape),
        ],
        out_specs=[
            pl.BlockSpec((BE, GC), lambda i: (i, 0)),
            pl.BlockSpec((BE, GC), lambda i: (i, 0)),
        ],
        out_shape=[
            jax.ShapeDtypeStruct((E, GC), jnp.float32),
            jax.ShapeDtypeStruct((E, GC), jnp.float32),
        ],
    )(edge_attr, w1e, b1r, W2, b2r, We1, be1r, We2, be2r, We3, be3r)

    cnts2 = _scatter_cnts(row)
    sums2 = _scatter_sums(eo, row)

    node_final_p = pl.pallas_call(
        _node_block_kernel,
        grid=(NP // BN,),
        in_specs=[
            pl.BlockSpec((NCORE, BN, GC), lambda i: (0, i, 0)),
            pl.BlockSpec((NCORE, BN, GC), lambda i: (0, i, 0)),
            full(Wn1.shape), full(bn1r.shape),
            full(Wn2.shape), full(bn2r.shape),
            full(Wn3.shape), full(bn3r.shape),
        ],
        out_specs=pl.BlockSpec((BN, GC), lambda i: (i, 0)),
        out_shape=jax.ShapeDtypeStruct((NP, GC), jnp.float32),
    )(sums2, cnts2, Wn1, bn1r, Wn2, bn2r, Wn3, bn3r)

    return node_final_p[:N], edge_final


# speedup vs baseline: 7.5630x; 1.8527x over previous
import functools

import jax
import jax.numpy as jnp
from jax import lax
from jax.experimental import pallas as pl
from jax.experimental.pallas import tpu as pltpu
from jax.experimental.pallas import tpu_sc as plsc

N = 10000
E = 320000
NP = 10240
GC = 128

NCORE = 2
NSUB = 16
NW = NCORE * NSUB
EPW = E // NW
CH = 80
NCHUNK = EPW // CH

BE = 2560
BN = 1280


def _leaky(x):
    return jnp.where(x > 0, x, 0.2 * x)


def _edge_block_kernel(eat, w1e, b1, w2, b2, we1, be1, we2, be2, we3, be3,
                       eo_ref, ef_ref):
    h = jnp.maximum(
        jax.lax.dot_general(eat[...], w1e[...], (((0,), (0,)), ((), ())),
                            preferred_element_type=jnp.float32)
        + b1[...], 0.0)
    eo = jnp.dot(h, w2[...], preferred_element_type=jnp.float32) + b2[...]
    eo_ref[...] = eo
    e1 = _leaky(eo)
    h1 = _leaky(jnp.dot(e1, we1[...], preferred_element_type=jnp.float32)
                + be1[...])
    h2 = _leaky(jnp.dot(h1, we2[...], preferred_element_type=jnp.float32)
                + be2[...])
    ef_ref[...] = (jnp.dot(h2, we3[...], preferred_element_type=jnp.float32)
                   + be3[...])


def _node_block_kernel(sums, cnts, wn1, bn1, wn2, bn2, wn3, bn3, out_ref):
    s = sums[0] + sums[1]
    c = cnts[0, :, 0:1] + cnts[1, :, 0:1]
    nod = s / jnp.maximum(c, 1.0)
    n1 = _leaky(nod)
    h1 = _leaky(jnp.dot(n1, wn1[...], preferred_element_type=jnp.float32)
                + bn1[...])
    h2 = _leaky(jnp.dot(h1, wn2[...], preferred_element_type=jnp.float32)
                + bn2[...])
    out_ref[...] = (jnp.dot(h2, wn3[...], preferred_element_type=jnp.float32)
                    + bn3[...])


def _scatter_sums_body(eo_hbm, row_hbm, sums_out,
                       idx0, idx1, rows0, rows1, zrow_v, sum_sh,
                       sem_i0, sem_i1, sem_r0, sem_r1):
    c = lax.axis_index("c")
    s = lax.axis_index("s")
    wid = s * NCORE + c

    zeros16 = jnp.zeros((16,), jnp.float32)
    for i in range(16):
        for j in range(GC // 16):
            zrow_v[i, 16 * j:16 * (j + 1)] = zeros16

    stripe = NP // NSUB
    nb = stripe // 16

    def _zero_blk(i, carry):
        base = s * stripe + i * 16
        pltpu.sync_copy(zrow_v, sum_sh.at[pl.ds(base, 16)])
        return carry

    lax.fori_loop(0, nb, _zero_blk, 0)
    plsc.subcore_barrier()

    ebase = wid * EPW

    def _issue(k, idx_b, rows_b, sem_i, sem_r):
        base = ebase + k * CH
        pltpu.async_copy(row_hbm.at[pl.ds(base, CH)], idx_b, sem_i)
        pltpu.async_copy(eo_hbm.at[pl.ds(base, CH)], rows_b, sem_r)

    def _wait(idx_b, rows_b, sem_i, sem_r):
        pltpu.make_async_copy(row_hbm.at[pl.ds(ebase, CH)], idx_b,
                              sem_i).wait()
        pltpu.make_async_copy(eo_hbm.at[pl.ds(ebase, CH)], rows_b,
                              sem_r).wait()

    _issue(0, idx0, rows0, sem_i0, sem_r0)

    def _pair(j2, carry):
        k0 = 2 * j2
        _issue(k0 + 1, idx1, rows1, sem_i1, sem_r1)
        _wait(idx0, rows0, sem_i0, sem_r0)
        pltpu.sync_copy(rows0, sum_sh.at[idx0], add=True)
        _issue(k0 + 2, idx0, rows0, sem_i0, sem_r0)
        _wait(idx1, rows1, sem_i1, sem_r1)
        pltpu.sync_copy(rows1, sum_sh.at[idx1], add=True)
        return carry

    lax.fori_loop(0, (NCHUNK - 1) // 2, _pair, 0)
    _wait(idx0, rows0, sem_i0, sem_r0)
    pltpu.sync_copy(rows0, sum_sh.at[idx0], add=True)
    plsc.subcore_barrier()

    def _out_blk(i, carry):
        base = s * stripe + i * 16
        pltpu.sync_copy(sum_sh.at[pl.ds(base, 16)],
                        sums_out.at[c, pl.ds(base, 16)])
        return carry

    lax.fori_loop(0, nb, _out_blk, 0)


def _scatter_cnts_body(row_hbm, cnts_out, idx0, idx1, ones_v, zrow_v, cnt_sh,
                       sem_i0, sem_i1):
    c = lax.axis_index("c")
    s = lax.axis_index("s")
    wid = s * NCORE + c

    zeros16 = jnp.zeros((16,), jnp.float32)
    ones16 = jnp.ones((16,), jnp.float32)
    for i in range(16):
        for j in range(GC // 16):
            zrow_v[i, 16 * j:16 * (j + 1)] = zeros16

    def _init_ones(i, carry):
        for j in range(GC // 16):
            ones_v[i, 16 * j:16 * (j + 1)] = ones16
        return carry

    lax.fori_loop(0, CH, _init_ones, 0)

    stripe = NP // NSUB
    nb = stripe // 16

    def _zero_blk(i, carry):
        base = s * stripe + i * 16
        pltpu.sync_copy(zrow_v, cnt_sh.at[pl.ds(base, 16)])
        return carry

    lax.fori_loop(0, nb, _zero_blk, 0)
    plsc.subcore_barrier()

    ebase = wid * EPW

    def _issue(k, idx_b, sem_i):
        pltpu.async_copy(row_hbm.at[pl.ds(ebase + k * CH, CH)], idx_b, sem_i)

    def _wait(idx_b, sem_i):
        pltpu.make_async_copy(row_hbm.at[pl.ds(ebase, CH)], idx_b,
                              sem_i).wait()

    _issue(0, idx0, sem_i0)

    def _pair(j2, carry):
        k0 = 2 * j2
        _issue(k0 + 1, idx1, sem_i1)
        _wait(idx0, sem_i0)
        pltpu.sync_copy(ones_v, cnt_sh.at[idx0], add=True)
        _issue(k0 + 2, idx0, sem_i0)
        _wait(idx1, sem_i1)
        pltpu.sync_copy(ones_v, cnt_sh.at[idx1], add=True)
        return carry

    lax.fori_loop(0, (NCHUNK - 1) // 2, _pair, 0)
    _wait(idx0, sem_i0)
    pltpu.sync_copy(ones_v, cnt_sh.at[idx0], add=True)
    plsc.subcore_barrier()

    def _out_blk(i, carry):
        base = s * stripe + i * 16
        pltpu.sync_copy(cnt_sh.at[pl.ds(base, 16)],
                        cnts_out.at[c, pl.ds(base, 16)])
        return carry

    lax.fori_loop(0, nb, _out_blk, 0)


_MESH = plsc.VectorSubcoreMesh(core_axis_name="c", subcore_axis_name="s",
                               num_cores=NCORE, num_subcores=NSUB)

_scatter_sums = functools.partial(
    pl.kernel,
    out_type=jax.ShapeDtypeStruct((NCORE, NP, GC), jnp.float32),
    mesh=_MESH,
    scratch_types=[
        pltpu.VMEM((CH,), jnp.int32),
        pltpu.VMEM((CH,), jnp.int32),
        pltpu.VMEM((CH, GC), jnp.float32),
        pltpu.VMEM((CH, GC), jnp.float32),
        pltpu.VMEM((16, GC), jnp.float32),
        pltpu.VMEM_SHARED((NP, GC), jnp.float32),
        pltpu.SemaphoreType.DMA,
        pltpu.SemaphoreType.DMA,
        pltpu.SemaphoreType.DMA,
        pltpu.SemaphoreType.DMA,
    ],
)(_scatter_sums_body)

_scatter_cnts = functools.partial(
    pl.kernel,
    out_type=jax.ShapeDtypeStruct((NCORE, NP, GC), jnp.float32),
    mesh=_MESH,
    scratch_types=[
        pltpu.VMEM((CH,), jnp.int32),
        pltpu.VMEM((CH,), jnp.int32),
        pltpu.VMEM((CH, GC), jnp.float32),
        pltpu.VMEM((16, GC), jnp.float32),
        pltpu.VMEM_SHARED((NP, GC), jnp.float32),
        pltpu.SemaphoreType.DMA,
        pltpu.SemaphoreType.DMA,
    ],
)(_scatter_cnts_body)


def kernel(node_attr, edge_attr, edge_index, W1, b1, W2, b2,
           Wn1, bn1, Wn2, bn2, Wn3, bn3,
           We1, be1, We2, be2, We3, be3):
    nc = node_attr.shape[1]
    w1e = W1[2 * nc:]
    b1r = b1.reshape(1, -1)
    b2r = b2.reshape(1, -1)
    bn1r, bn2r, bn3r = bn1.reshape(1, -1), bn2.reshape(1, -1), bn3.reshape(1, -1)
    be1r, be2r, be3r = be1.reshape(1, -1), be2.reshape(1, -1), be3.reshape(1, -1)
    row = edge_index[0]

    eat = edge_attr.T

    full = lambda shape: pl.BlockSpec(shape, lambda i: (0,) * len(shape))
    eo, edge_final = pl.pallas_call(
        _edge_block_kernel,
        grid=(E // BE,),
        in_specs=[
            pl.BlockSpec((edge_attr.shape[1], BE), lambda i: (0, i)),
            full(w1e.shape), full(b1r.shape),
            full(W2.shape), full(b2r.shape),
            full(We1.shape), full(be1r.shape),
            full(We2.shape), full(be2r.shape),
            full(We3.shape), full(be3r.shape),
        ],
        out_specs=[
            pl.BlockSpec((BE, GC), lambda i: (i, 0)),
            pl.BlockSpec((BE, GC), lambda i: (i, 0)),
        ],
        out_shape=[
            jax.ShapeDtypeStruct((E, GC), jnp.float32),
            jax.ShapeDtypeStruct((E, GC), jnp.float32),
        ],
    )(eat, w1e, b1r, W2, b2r, We1, be1r, We2, be2r, We3, be3r)

    cnts2 = _scatter_cnts(row)
    sums2 = _scatter_sums(eo, row)

    node_final_p = pl.pallas_call(
        _node_block_kernel,
        grid=(NP // BN,),
        in_specs=[
            pl.BlockSpec((NCORE, BN, GC), lambda i: (0, i, 0)),
            pl.BlockSpec((NCORE, BN, GC), lambda i: (0, i, 0)),
            full(Wn1.shape), full(bn1r.shape),
            full(Wn2.shape), full(bn2r.shape),
            full(Wn3.shape), full(bn3r.shape),
        ],
        out_specs=pl.BlockSpec((BN, GC), lambda i: (i, 0)),
        out_shape=jax.ShapeDtypeStruct((NP, GC), jnp.float32),
    )(sums2, cnts2, Wn1, bn1r, Wn2, bn2r, Wn3, bn3r)

    return node_final_p[:N], edge_final


# speedup vs baseline: 7.6744x; 1.0147x over previous
import functools

import jax
import jax.numpy as jnp
from jax import lax
from jax.experimental import pallas as pl
from jax.experimental.pallas import tpu as pltpu
from jax.experimental.pallas import tpu_sc as plsc

N = 10000
E = 320000
NP = 10240
GC = 128

NCORE = 2
NSUB = 16
NW = NCORE * NSUB
EPW = E // NW
CH = 80
NCHUNK = EPW // CH

BE = 2560
BN = 1000


def _leaky(x):
    return jnp.where(x > 0, x, 0.2 * x)


def _edge_block_kernel(eat, w1e, b1, w2, b2, we1, be1, we2, be2, we3, be3,
                       eo_ref, ef_ref):
    h = jnp.maximum(
        jax.lax.dot_general(eat[...], w1e[...], (((0,), (0,)), ((), ())),
                            preferred_element_type=jnp.float32)
        + b1[...], 0.0)
    eo = jnp.dot(h, w2[...], preferred_element_type=jnp.float32) + b2[...]
    eo_ref[...] = eo
    e1 = _leaky(eo).astype(jnp.bfloat16)
    h1 = _leaky(jnp.dot(e1, we1[...], preferred_element_type=jnp.float32)
                + be1[...]).astype(jnp.bfloat16)
    h2 = _leaky(jnp.dot(h1, we2[...], preferred_element_type=jnp.float32)
                + be2[...]).astype(jnp.bfloat16)
    ef_ref[...] = (jnp.dot(h2, we3[...], preferred_element_type=jnp.float32)
                   + be3[...])


def _node_block_kernel(sums, cnts, wn1, bn1, wn2, bn2, wn3, bn3, out_ref):
    s = sums[0] + sums[1]
    c = cnts[0, :, 0:1] + cnts[1, :, 0:1]
    nod = s / jnp.maximum(c, 1.0)
    n1 = _leaky(nod)
    h1 = _leaky(jnp.dot(n1, wn1[...], preferred_element_type=jnp.float32)
                + bn1[...])
    h2 = _leaky(jnp.dot(h1, wn2[...], preferred_element_type=jnp.float32)
                + bn2[...])
    out_ref[...] = (jnp.dot(h2, wn3[...], preferred_element_type=jnp.float32)
                    + bn3[...])


def _scatter_sums_body(eo_hbm, ei_hbm, sums_out,
                       idx0, idx1, rows0, rows1, zrow_v, sum_sh,
                       sem_i0, sem_i1, sem_r0, sem_r1):
    c = lax.axis_index("c")
    s = lax.axis_index("s")
    wid = s * NCORE + c

    zeros16 = jnp.zeros((16,), jnp.float32)
    for i in range(16):
        for j in range(GC // 16):
            zrow_v[i, 16 * j:16 * (j + 1)] = zeros16

    stripe = NP // NSUB
    nb = stripe // 16

    def _zero_blk(i, carry):
        base = s * stripe + i * 16
        pltpu.sync_copy(zrow_v, sum_sh.at[pl.ds(base, 16)])
        return carry

    lax.fori_loop(0, nb, _zero_blk, 0)
    plsc.subcore_barrier()

    ebase = wid * EPW

    def _issue(k, idx_b, rows_b, sem_i, sem_r):
        base = ebase + k * CH
        pltpu.async_copy(ei_hbm.at[pl.ds(base, CH)], idx_b, sem_i)
        pltpu.async_copy(eo_hbm.at[pl.ds(base, CH)], rows_b, sem_r)

    def _wait(idx_b, rows_b, sem_i, sem_r):
        pltpu.make_async_copy(ei_hbm.at[pl.ds(ebase, CH)], idx_b,
                              sem_i).wait()
        pltpu.make_async_copy(eo_hbm.at[pl.ds(ebase, CH)], rows_b,
                              sem_r).wait()

    _issue(0, idx0, rows0, sem_i0, sem_r0)

    def _pair(j2, carry):
        k0 = 2 * j2
        _issue(k0 + 1, idx1, rows1, sem_i1, sem_r1)
        _wait(idx0, rows0, sem_i0, sem_r0)
        pltpu.sync_copy(rows0, sum_sh.at[idx0], add=True)
        _issue(k0 + 2, idx0, rows0, sem_i0, sem_r0)
        _wait(idx1, rows1, sem_i1, sem_r1)
        pltpu.sync_copy(rows1, sum_sh.at[idx1], add=True)
        return carry

    lax.fori_loop(0, (NCHUNK - 1) // 2, _pair, 0)
    _wait(idx0, rows0, sem_i0, sem_r0)
    pltpu.sync_copy(rows0, sum_sh.at[idx0], add=True)
    plsc.subcore_barrier()

    def _out_blk(i, carry):
        base = s * stripe + i * 16
        pltpu.sync_copy(sum_sh.at[pl.ds(base, 16)],
                        sums_out.at[c, pl.ds(base, 16)])
        return carry

    lax.fori_loop(0, nb, _out_blk, 0)


def _scatter_cnts_body(ei_hbm, cnts_out, idx0, idx1, ones_v, zrow_v, cnt_sh,
                       sem_i0, sem_i1):
    c = lax.axis_index("c")
    s = lax.axis_index("s")
    wid = s * NCORE + c

    zeros16 = jnp.zeros((16,), jnp.float32)
    ones16 = jnp.ones((16,), jnp.float32)
    for i in range(16):
        for j in range(GC // 16):
            zrow_v[i, 16 * j:16 * (j + 1)] = zeros16

    def _init_ones(i, carry):
        for j in range(GC // 16):
            ones_v[i, 16 * j:16 * (j + 1)] = ones16
        return carry

    lax.fori_loop(0, CH, _init_ones, 0)

    stripe = NP // NSUB
    nb = stripe // 16

    def _zero_blk(i, carry):
        base = s * stripe + i * 16
        pltpu.sync_copy(zrow_v, cnt_sh.at[pl.ds(base, 16)])
        return carry

    lax.fori_loop(0, nb, _zero_blk, 0)
    plsc.subcore_barrier()

    ebase = wid * EPW

    def _issue(k, idx_b, sem_i):
        pltpu.async_copy(ei_hbm.at[pl.ds(ebase + k * CH, CH)], idx_b, sem_i)

    def _wait(idx_b, sem_i):
        pltpu.make_async_copy(ei_hbm.at[pl.ds(ebase, CH)], idx_b,
                              sem_i).wait()

    _issue(0, idx0, sem_i0)

    def _pair(j2, carry):
        k0 = 2 * j2
        _issue(k0 + 1, idx1, sem_i1)
        _wait(idx0, sem_i0)
        pltpu.sync_copy(ones_v, cnt_sh.at[idx0], add=True)
        _issue(k0 + 2, idx0, sem_i0)
        _wait(idx1, sem_i1)
        pltpu.sync_copy(ones_v, cnt_sh.at[idx1], add=True)
        return carry

    lax.fori_loop(0, (NCHUNK - 1) // 2, _pair, 0)
    _wait(idx0, sem_i0)
    pltpu.sync_copy(ones_v, cnt_sh.at[idx0], add=True)
    plsc.subcore_barrier()

    def _out_blk(i, carry):
        base = s * stripe + i * 16
        pltpu.sync_copy(cnt_sh.at[pl.ds(base, 16)],
                        cnts_out.at[c, pl.ds(base, 16)])
        return carry

    lax.fori_loop(0, nb, _out_blk, 0)


_MESH = plsc.VectorSubcoreMesh(core_axis_name="c", subcore_axis_name="s",
                               num_cores=NCORE, num_subcores=NSUB)

_scatter_sums = functools.partial(
    pl.kernel,
    out_type=jax.ShapeDtypeStruct((NCORE, NP, GC), jnp.float32),
    mesh=_MESH,
    scratch_types=[
        pltpu.VMEM((CH,), jnp.int32),
        pltpu.VMEM((CH,), jnp.int32),
        pltpu.VMEM((CH, GC), jnp.float32),
        pltpu.VMEM((CH, GC), jnp.float32),
        pltpu.VMEM((16, GC), jnp.float32),
        pltpu.VMEM_SHARED((NP, GC), jnp.float32),
        pltpu.SemaphoreType.DMA,
        pltpu.SemaphoreType.DMA,
        pltpu.SemaphoreType.DMA,
        pltpu.SemaphoreType.DMA,
    ],
)(_scatter_sums_body)

_scatter_cnts = functools.partial(
    pl.kernel,
    out_type=jax.ShapeDtypeStruct((NCORE, NP, GC), jnp.float32),
    mesh=_MESH,
    scratch_types=[
        pltpu.VMEM((CH,), jnp.int32),
        pltpu.VMEM((CH,), jnp.int32),
        pltpu.VMEM((CH, GC), jnp.float32),
        pltpu.VMEM((16, GC), jnp.float32),
        pltpu.VMEM_SHARED((NP, GC), jnp.float32),
        pltpu.SemaphoreType.DMA,
        pltpu.SemaphoreType.DMA,
    ],
)(_scatter_cnts_body)


def kernel(node_attr, edge_attr, edge_index, W1, b1, W2, b2,
           Wn1, bn1, Wn2, bn2, Wn3, bn3,
           We1, be1, We2, be2, We3, be3):
    nc = node_attr.shape[1]
    w1e = W1[2 * nc:]
    b1r = b1.reshape(1, -1)
    b2r = b2.reshape(1, -1)
    bn1r, bn2r, bn3r = bn1.reshape(1, -1), bn2.reshape(1, -1), bn3.reshape(1, -1)
    be1r, be2r, be3r = be1.reshape(1, -1), be2.reshape(1, -1), be3.reshape(1, -1)
    we1b = We1.astype(jnp.bfloat16)
    we2b = We2.astype(jnp.bfloat16)
    we3b = We3.astype(jnp.bfloat16)

    eat = edge_attr.T

    full = lambda shape: pl.BlockSpec(shape, lambda i: (0,) * len(shape))
    eo, edge_final = pl.pallas_call(
        _edge_block_kernel,
        grid=(E // BE,),
        in_specs=[
            pl.BlockSpec((edge_attr.shape[1], BE), lambda i: (0, i)),
            full(w1e.shape), full(b1r.shape),
            full(W2.shape), full(b2r.shape),
            full(We1.shape), full(be1r.shape),
            full(We2.shape), full(be2r.shape),
            full(We3.shape), full(be3r.shape),
        ],
        out_specs=[
            pl.BlockSpec((BE, GC), lambda i: (i, 0)),
            pl.BlockSpec((BE, GC), lambda i: (i, 0)),
        ],
        out_shape=[
            jax.ShapeDtypeStruct((E, GC), jnp.float32),
            jax.ShapeDtypeStruct((E, GC), jnp.float32),
        ],
    )(eat, w1e, b1r, W2, b2r, we1b, be1r, we2b, be2r, we3b, be3r)

    ei_flat = edge_index.reshape(-1)
    cnts2 = _scatter_cnts(ei_flat)
    sums2 = _scatter_sums(eo, ei_flat)

    node_final = pl.pallas_call(
        _node_block_kernel,
        grid=(N // BN,),
        in_specs=[
            pl.BlockSpec((NCORE, BN, GC), lambda i: (0, i, 0)),
            pl.BlockSpec((NCORE, BN, GC), lambda i: (0, i, 0)),
            full(Wn1.shape), full(bn1r.shape),
            full(Wn2.shape), full(bn2r.shape),
            full(Wn3.shape), full(bn3r.shape),
        ],
        out_specs=pl.BlockSpec((BN, GC), lambda i: (i, 0)),
        out_shape=jax.ShapeDtypeStruct((N, GC), jnp.float32),
    )(sums2, cnts2, Wn1, bn1r, Wn2, bn2r, Wn3, bn3r)

    return node_final, edge_final


# speedup vs baseline: 8.4501x; 1.1011x over previous
import functools

import jax
import jax.numpy as jnp
from jax import lax
from jax.experimental import pallas as pl
from jax.experimental.pallas import tpu as pltpu
from jax.experimental.pallas import tpu_sc as plsc

N = 10000
E = 320000
NP = 10240
GC = 128

NCORE = 2
NSUB = 16
NW = NCORE * NSUB
EPW = E // NW
CH = 80
NCHUNK = EPW // CH

BE = 2560
BN = 1000


def _leaky(x):
    return jnp.where(x > 0, x, 0.2 * x)


def _edge_out(eat, w1e, b1, w2, b2):
    h = jnp.maximum(
        jax.lax.dot_general(eat[...], w1e[...], (((0,), (0,)), ((), ())),
                            preferred_element_type=jnp.float32)
        + b1[...], 0.0)
    return jnp.dot(h, w2[...], preferred_element_type=jnp.float32) + b2[...]


def _edge_eo_kernel(eat, w1e, b1, w2, b2, eo_ref):
    eo_ref[...] = _edge_out(eat, w1e, b1, w2, b2)


def _edge_ef_kernel(eat, w1e, b1, w2, b2, we1, be1, we2, be2, we3, be3,
                    ef_ref):
    eo = _edge_out(eat, w1e, b1, w2, b2)
    e1 = _leaky(eo).astype(jnp.bfloat16)
    h1 = _leaky(jnp.dot(e1, we1[...], preferred_element_type=jnp.float32)
                + be1[...]).astype(jnp.bfloat16)
    h2 = _leaky(jnp.dot(h1, we2[...], preferred_element_type=jnp.float32)
                + be2[...]).astype(jnp.bfloat16)
    ef_ref[...] = (jnp.dot(h2, we3[...], preferred_element_type=jnp.float32)
                   + be3[...])


def _node_block_kernel(sums, cnts, wn1, bn1, wn2, bn2, wn3, bn3, out_ref):
    s = sums[0] + sums[1]
    c = cnts[0, :, 0:1] + cnts[1, :, 0:1]
    nod = s / jnp.maximum(c, 1.0)
    n1 = _leaky(nod)
    h1 = _leaky(jnp.dot(n1, wn1[...], preferred_element_type=jnp.float32)
                + bn1[...])
    h2 = _leaky(jnp.dot(h1, wn2[...], preferred_element_type=jnp.float32)
                + bn2[...])
    out_ref[...] = (jnp.dot(h2, wn3[...], preferred_element_type=jnp.float32)
                    + bn3[...])


def _scatter_sums_body(eo_hbm, ei_hbm, sums_out,
                       idx0, idx1, rows0, rows1, zrow_v, sum_sh,
                       sem_i0, sem_i1, sem_r0, sem_r1):
    c = lax.axis_index("c")
    s = lax.axis_index("s")
    wid = s * NCORE + c

    zeros16 = jnp.zeros((16,), jnp.float32)
    for i in range(16):
        for j in range(GC // 16):
            zrow_v[i, 16 * j:16 * (j + 1)] = zeros16

    stripe = NP // NSUB
    nb = stripe // 16

    def _zero_blk(i, carry):
        base = s * stripe + i * 16
        pltpu.sync_copy(zrow_v, sum_sh.at[pl.ds(base, 16)])
        return carry

    lax.fori_loop(0, nb, _zero_blk, 0)
    plsc.subcore_barrier()

    ebase = wid * EPW

    def _issue(k, idx_b, rows_b, sem_i, sem_r):
        base = ebase + k * CH
        pltpu.async_copy(ei_hbm.at[pl.ds(base, CH)], idx_b, sem_i)
        pltpu.async_copy(eo_hbm.at[pl.ds(base, CH)], rows_b, sem_r)

    def _wait(idx_b, rows_b, sem_i, sem_r):
        pltpu.make_async_copy(ei_hbm.at[pl.ds(ebase, CH)], idx_b,
                              sem_i).wait()
        pltpu.make_async_copy(eo_hbm.at[pl.ds(ebase, CH)], rows_b,
                              sem_r).wait()

    _issue(0, idx0, rows0, sem_i0, sem_r0)

    def _pair(j2, carry):
        k0 = 2 * j2
        _issue(k0 + 1, idx1, rows1, sem_i1, sem_r1)
        _wait(idx0, rows0, sem_i0, sem_r0)
        pltpu.sync_copy(rows0, sum_sh.at[idx0], add=True)
        _issue(k0 + 2, idx0, rows0, sem_i0, sem_r0)
        _wait(idx1, rows1, sem_i1, sem_r1)
        pltpu.sync_copy(rows1, sum_sh.at[idx1], add=True)
        return carry

    lax.fori_loop(0, (NCHUNK - 1) // 2, _pair, 0)
    _wait(idx0, rows0, sem_i0, sem_r0)
    pltpu.sync_copy(rows0, sum_sh.at[idx0], add=True)
    plsc.subcore_barrier()

    def _out_blk(i, carry):
        base = s * stripe + i * 16
        pltpu.sync_copy(sum_sh.at[pl.ds(base, 16)],
                        sums_out.at[c, pl.ds(base, 16)])
        return carry

    lax.fori_loop(0, nb, _out_blk, 0)


def _scatter_cnts_body(ei_hbm, cnts_out, idx0, idx1, ones_v, zrow_v, cnt_sh,
                       sem_i0, sem_i1):
    c = lax.axis_index("c")
    s = lax.axis_index("s")
    wid = s * NCORE + c

    zeros16 = jnp.zeros((16,), jnp.float32)
    ones16 = jnp.ones((16,), jnp.float32)
    for i in range(16):
        for j in range(GC // 16):
            zrow_v[i, 16 * j:16 * (j + 1)] = zeros16

    def _init_ones(i, carry):
        for j in range(GC // 16):
            ones_v[i, 16 * j:16 * (j + 1)] = ones16
        return carry

    lax.fori_loop(0, CH, _init_ones, 0)

    stripe = NP // NSUB
    nb = stripe // 16

    def _zero_blk(i, carry):
        base = s * stripe + i * 16
        pltpu.sync_copy(zrow_v, cnt_sh.at[pl.ds(base, 16)])
        return carry

    lax.fori_loop(0, nb, _zero_blk, 0)
    plsc.subcore_barrier()

    ebase = wid * EPW

    def _issue(k, idx_b, sem_i):
        pltpu.async_copy(ei_hbm.at[pl.ds(ebase + k * CH, CH)], idx_b, sem_i)

    def _wait(idx_b, sem_i):
        pltpu.make_async_copy(ei_hbm.at[pl.ds(ebase, CH)], idx_b,
                              sem_i).wait()

    _issue(0, idx0, sem_i0)

    def _pair(j2, carry):
        k0 = 2 * j2
        _issue(k0 + 1, idx1, sem_i1)
        _wait(idx0, sem_i0)
        pltpu.sync_copy(ones_v, cnt_sh.at[idx0], add=True)
        _issue(k0 + 2, idx0, sem_i0)
        _wait(idx1, sem_i1)
        pltpu.sync_copy(ones_v, cnt_sh.at[idx1], add=True)
        return carry

    lax.fori_loop(0, (NCHUNK - 1) // 2, _pair, 0)
    _wait(idx0, sem_i0)
    pltpu.sync_copy(ones_v, cnt_sh.at[idx0], add=True)
    plsc.subcore_barrier()

    def _out_blk(i, carry):
        base = s * stripe + i * 16
        pltpu.sync_copy(cnt_sh.at[pl.ds(base, 16)],
                        cnts_out.at[c, pl.ds(base, 16)])
        return carry

    lax.fori_loop(0, nb, _out_blk, 0)


_MESH = plsc.VectorSubcoreMesh(core_axis_name="c", subcore_axis_name="s",
                               num_cores=NCORE, num_subcores=NSUB)

_scatter_sums = functools.partial(
    pl.kernel,
    out_type=jax.ShapeDtypeStruct((NCORE, NP, GC), jnp.float32),
    mesh=_MESH,
    scratch_types=[
        pltpu.VMEM((CH,), jnp.int32),
        pltpu.VMEM((CH,), jnp.int32),
        pltpu.VMEM((CH, GC), jnp.float32),
        pltpu.VMEM((CH, GC), jnp.float32),
        pltpu.VMEM((16, GC), jnp.float32),
        pltpu.VMEM_SHARED((NP, GC), jnp.float32),
        pltpu.SemaphoreType.DMA,
        pltpu.SemaphoreType.DMA,
        pltpu.SemaphoreType.DMA,
        pltpu.SemaphoreType.DMA,
    ],
)(_scatter_sums_body)

_scatter_cnts = functools.partial(
    pl.kernel,
    out_type=jax.ShapeDtypeStruct((NCORE, NP, GC), jnp.float32),
    mesh=_MESH,
    scratch_types=[
        pltpu.VMEM((CH,), jnp.int32),
        pltpu.VMEM((CH,), jnp.int32),
        pltpu.VMEM((CH, GC), jnp.float32),
        pltpu.VMEM((16, GC), jnp.float32),
        pltpu.VMEM_SHARED((NP, GC), jnp.float32),
        pltpu.SemaphoreType.DMA,
        pltpu.SemaphoreType.DMA,
    ],
)(_scatter_cnts_body)


def kernel(node_attr, edge_attr, edge_index, W1, b1, W2, b2,
           Wn1, bn1, Wn2, bn2, Wn3, bn3,
           We1, be1, We2, be2, We3, be3):
    nc = node_attr.shape[1]
    w1e = W1[2 * nc:]
    b1r = b1.reshape(1, -1)
    b2r = b2.reshape(1, -1)
    bn1r, bn2r, bn3r = bn1.reshape(1, -1), bn2.reshape(1, -1), bn3.reshape(1, -1)
    be1r, be2r, be3r = be1.reshape(1, -1), be2.reshape(1, -1), be3.reshape(1, -1)
    we1b = We1.astype(jnp.bfloat16)
    we2b = We2.astype(jnp.bfloat16)
    we3b = We3.astype(jnp.bfloat16)

    eat = edge_attr.T

    full = lambda shape: pl.BlockSpec(shape, lambda i: (0,) * len(shape))
    eat_spec = pl.BlockSpec((edge_attr.shape[1], BE), lambda i: (0, i))
    eo = pl.pallas_call(
        _edge_eo_kernel,
        grid=(E // BE,),
        in_specs=[
            eat_spec,
            full(w1e.shape), full(b1r.shape),
            full(W2.shape), full(b2r.shape),
        ],
        out_specs=pl.BlockSpec((BE, GC), lambda i: (i, 0)),
        out_shape=jax.ShapeDtypeStruct((E, GC), jnp.float32),
    )(eat, w1e, b1r, W2, b2r)

    ei_flat = edge_index.reshape(-1)
    cnts2 = _scatter_cnts(ei_flat)
    sums2 = _scatter_sums(eo, ei_flat)

    edge_final = pl.pallas_call(
        _edge_ef_kernel,
        grid=(E // BE,),
        in_specs=[
            eat_spec,
            full(w1e.shape), full(b1r.shape),
            full(W2.shape), full(b2r.shape),
            full(We1.shape), full(be1r.shape),
            full(We2.shape), full(be2r.shape),
            full(We3.shape), full(be3r.shape),
        ],
        out_specs=pl.BlockSpec((BE, GC), lambda i: (i, 0)),
        out_shape=jax.ShapeDtypeStruct((E, GC), jnp.float32),
    )(eat, w1e, b1r, W2, b2r, we1b, be1r, we2b, be2r, we3b, be3r)

    node_final = pl.pallas_call(
        _node_block_kernel,
        grid=(N // BN,),
        in_specs=[
            pl.BlockSpec((NCORE, BN, GC), lambda i: (0, i, 0)),
            pl.BlockSpec((NCORE, BN, GC), lambda i: (0, i, 0)),
            full(Wn1.shape), full(bn1r.shape),
            full(Wn2.shape), full(bn2r.shape),
            full(Wn3.shape), full(bn3r.shape),
        ],
        out_specs=pl.BlockSpec((BN, GC), lambda i: (i, 0)),
        out_shape=jax.ShapeDtypeStruct((N, GC), jnp.float32),
    )(sums2, cnts2, Wn1, bn1r, Wn2, bn2r, Wn3, bn3r)

    return node_final, edge_final


# speedup vs baseline: 8.9487x; 1.0590x over previous
import functools

import jax
import jax.numpy as jnp
from jax import lax
from jax.experimental import pallas as pl
from jax.experimental.pallas import tpu as pltpu
from jax.experimental.pallas import tpu_sc as plsc

N = 10000
E = 320000
NP = 10240
GC = 128

NCORE = 2
NSUB = 16
NW = NCORE * NSUB
EPW = E // NW
CH = 80
NCHUNK = EPW // CH

BE = 2560
BN = 1000


def _leaky(x):
    return jnp.where(x > 0, x, 0.2 * x)


def _edge_out(eat, w1e, b1, w2, b2):
    h = jnp.maximum(
        jax.lax.dot_general(eat[...], w1e[...], (((0,), (0,)), ((), ())),
                            preferred_element_type=jnp.float32)
        + b1[...], 0.0).astype(jnp.bfloat16)
    return jnp.dot(h, w2[...], preferred_element_type=jnp.float32) + b2[...]


def _edge_eo_kernel(eat, w1e, b1, w2, b2, eo_ref):
    eo_ref[...] = _edge_out(eat, w1e, b1, w2, b2)


def _edge_ef_kernel(eat, w1e, b1, w2, b2, we1, be1, we2, be2, we3, be3,
                    ef_ref):
    eo = _edge_out(eat, w1e, b1, w2, b2)
    e1 = _leaky(eo).astype(jnp.bfloat16)
    h1 = _leaky(jnp.dot(e1, we1[...], preferred_element_type=jnp.float32)
                + be1[...]).astype(jnp.bfloat16)
    h2 = _leaky(jnp.dot(h1, we2[...], preferred_element_type=jnp.float32)
                + be2[...]).astype(jnp.bfloat16)
    ef_ref[...] = (jnp.dot(h2, we3[...], preferred_element_type=jnp.float32)
                   + be3[...])


def _node_block_kernel(sums, cnts, wn1, bn1, wn2, bn2, wn3, bn3, out_ref):
    s = sums[0] + sums[1]
    c = cnts[0, :, 0:1] + cnts[1, :, 0:1]
    nod = s / jnp.maximum(c, 1.0)
    n1 = _leaky(nod)
    h1 = _leaky(jnp.dot(n1, wn1[...], preferred_element_type=jnp.float32)
                + bn1[...])
    h2 = _leaky(jnp.dot(h1, wn2[...], preferred_element_type=jnp.float32)
                + bn2[...])
    out_ref[...] = (jnp.dot(h2, wn3[...], preferred_element_type=jnp.float32)
                    + bn3[...])


def _scatter_sums_body(eo_hbm, ei_hbm, sums_out,
                       idx0, idx1, idx2, idx3, rows0, rows1, rows2, rows3,
                       zrow_v, sum_sh,
                       sem_i0, sem_i1, sem_i2, sem_i3,
                       sem_r0, sem_r1, sem_r2, sem_r3):
    c = lax.axis_index("c")
    s = lax.axis_index("s")
    wid = s * NCORE + c

    idxs = (idx0, idx1, idx2, idx3)
    rows = (rows0, rows1, rows2, rows3)
    sems_i = (sem_i0, sem_i1, sem_i2, sem_i3)
    sems_r = (sem_r0, sem_r1, sem_r2, sem_r3)

    zeros16 = jnp.zeros((16,), jnp.float32)
    for i in range(16):
        for j in range(GC // 16):
            zrow_v[i, 16 * j:16 * (j + 1)] = zeros16

    stripe = NP // NSUB
    nb = stripe // 16

    def _zero_blk(i, carry):
        base = s * stripe + i * 16
        pltpu.sync_copy(zrow_v, sum_sh.at[pl.ds(base, 16)])
        return carry

    lax.fori_loop(0, nb, _zero_blk, 0)
    plsc.subcore_barrier()

    ebase = wid * EPW

    def _issue(k, b):
        base = ebase + k * CH
        pltpu.async_copy(ei_hbm.at[pl.ds(base, CH)], idxs[b], sems_i[b])
        pltpu.async_copy(eo_hbm.at[pl.ds(base, CH)], rows[b], sems_r[b])

    def _wait(b):
        pltpu.make_async_copy(ei_hbm.at[pl.ds(ebase, CH)], idxs[b],
                              sems_i[b]).wait()
        pltpu.make_async_copy(eo_hbm.at[pl.ds(ebase, CH)], rows[b],
                              sems_r[b]).wait()

    for b in range(3):
        _issue(b, b)

    def _quad(j4, carry):
        k0 = 4 * j4
        for b in range(4):
            k = k0 + b
            _wait(b)
            pltpu.sync_copy(rows[b], sum_sh.at[idxs[b]], add=True)

            @pl.when(k + 3 <= NCHUNK - 1)
            def _():
                _issue(k + 3, (b + 3) % 4)
        return carry

    lax.fori_loop(0, (NCHUNK - 1) // 4, _quad, 0)
    _wait(0)
    pltpu.sync_copy(rows[0], sum_sh.at[idxs[0]], add=True)
    plsc.subcore_barrier()

    def _out_blk(i, carry):
        base = s * stripe + i * 16
        pltpu.sync_copy(sum_sh.at[pl.ds(base, 16)],
                        sums_out.at[c, pl.ds(base, 16)])
        return carry

    lax.fori_loop(0, nb, _out_blk, 0)


def _scatter_cnts_body(ei_hbm, cnts_out, idx0, idx1, ones_v, zrow_v, cnt_sh,
                       sem_i0, sem_i1):
    c = lax.axis_index("c")
    s = lax.axis_index("s")
    wid = s * NCORE + c

    zeros16 = jnp.zeros((16,), jnp.float32)
    ones16 = jnp.ones((16,), jnp.float32)
    for i in range(16):
        for j in range(GC // 16):
            zrow_v[i, 16 * j:16 * (j + 1)] = zeros16

    def _init_ones(i, carry):
        for j in range(GC // 16):
            ones_v[i, 16 * j:16 * (j + 1)] = ones16
        return carry

    lax.fori_loop(0, CH, _init_ones, 0)

    stripe = NP // NSUB
    nb = stripe // 16

    def _zero_blk(i, carry):
        base = s * stripe + i * 16
        pltpu.sync_copy(zrow_v, cnt_sh.at[pl.ds(base, 16)])
        return carry

    lax.fori_loop(0, nb, _zero_blk, 0)
    plsc.subcore_barrier()

    ebase = wid * EPW

    def _issue(k, idx_b, sem_i):
        pltpu.async_copy(ei_hbm.at[pl.ds(ebase + k * CH, CH)], idx_b, sem_i)

    def _wait(idx_b, sem_i):
        pltpu.make_async_copy(ei_hbm.at[pl.ds(ebase, CH)], idx_b,
                              sem_i).wait()

    _issue(0, idx0, sem_i0)

    def _pair(j2, carry):
        k0 = 2 * j2
        _issue(k0 + 1, idx1, sem_i1)
        _wait(idx0, sem_i0)
        pltpu.sync_copy(ones_v, cnt_sh.at[idx0], add=True)
        _issue(k0 + 2, idx0, sem_i0)
        _wait(idx1, sem_i1)
        pltpu.sync_copy(ones_v, cnt_sh.at[idx1], add=True)
        return carry

    lax.fori_loop(0, (NCHUNK - 1) // 2, _pair, 0)
    _wait(idx0, sem_i0)
    pltpu.sync_copy(ones_v, cnt_sh.at[idx0], add=True)
    plsc.subcore_barrier()

    def _out_blk(i, carry):
        base = s * stripe + i * 16
        pltpu.sync_copy(cnt_sh.at[pl.ds(base, 16)],
                        cnts_out.at[c, pl.ds(base, 16)])
        return carry

    lax.fori_loop(0, nb, _out_blk, 0)


_MESH = plsc.VectorSubcoreMesh(core_axis_name="c", subcore_axis_name="s",
                               num_cores=NCORE, num_subcores=NSUB)

_scatter_sums = functools.partial(
    pl.kernel,
    out_type=jax.ShapeDtypeStruct((NCORE, NP, GC), jnp.float32),
    mesh=_MESH,
    scratch_types=[
        pltpu.VMEM((CH,), jnp.int32),
        pltpu.VMEM((CH,), jnp.int32),
        pltpu.VMEM((CH,), jnp.int32),
        pltpu.VMEM((CH,), jnp.int32),
        pltpu.VMEM((CH, GC), jnp.float32),
        pltpu.VMEM((CH, GC), jnp.float32),
        pltpu.VMEM((CH, GC), jnp.float32),
        pltpu.VMEM((CH, GC), jnp.float32),
        pltpu.VMEM((16, GC), jnp.float32),
        pltpu.VMEM_SHARED((NP, GC), jnp.float32),
        pltpu.SemaphoreType.DMA,
        pltpu.SemaphoreType.DMA,
        pltpu.SemaphoreType.DMA,
        pltpu.SemaphoreType.DMA,
        pltpu.SemaphoreType.DMA,
        pltpu.SemaphoreType.DMA,
        pltpu.SemaphoreType.DMA,
        pltpu.SemaphoreType.DMA,
    ],
)(_scatter_sums_body)

_scatter_cnts = functools.partial(
    pl.kernel,
    out_type=jax.ShapeDtypeStruct((NCORE, NP, GC), jnp.float32),
    mesh=_MESH,
    scratch_types=[
        pltpu.VMEM((CH,), jnp.int32),
        pltpu.VMEM((CH,), jnp.int32),
        pltpu.VMEM((CH, GC), jnp.float32),
        pltpu.VMEM((16, GC), jnp.float32),
        pltpu.VMEM_SHARED((NP, GC), jnp.float32),
        pltpu.SemaphoreType.DMA,
        pltpu.SemaphoreType.DMA,
    ],
)(_scatter_cnts_body)


def kernel(node_attr, edge_attr, edge_index, W1, b1, W2, b2,
           Wn1, bn1, Wn2, bn2, Wn3, bn3,
           We1, be1, We2, be2, We3, be3):
    nc = node_attr.shape[1]
    w1e = W1[2 * nc:]
    b1r = b1.reshape(1, -1)
    b2r = b2.reshape(1, -1)
    bn1r, bn2r, bn3r = bn1.reshape(1, -1), bn2.reshape(1, -1), bn3.reshape(1, -1)
    be1r, be2r, be3r = be1.reshape(1, -1), be2.reshape(1, -1), be3.reshape(1, -1)
    w2b = W2.astype(jnp.bfloat16)
    we1b = We1.astype(jnp.bfloat16)
    we2b = We2.astype(jnp.bfloat16)
    we3b = We3.astype(jnp.bfloat16)

    eat = edge_attr.T

    full = lambda shape: pl.BlockSpec(shape, lambda i: (0,) * len(shape))
    eat_spec = pl.BlockSpec((edge_attr.shape[1], BE), lambda i: (0, i))
    eo = pl.pallas_call(
        _edge_eo_kernel,
        grid=(E // BE,),
        in_specs=[
            eat_spec,
            full(w1e.shape), full(b1r.shape),
            full(W2.shape), full(b2r.shape),
        ],
        out_specs=pl.BlockSpec((BE, GC), lambda i: (i, 0)),
        out_shape=jax.ShapeDtypeStruct((E, GC), jnp.float32),
    )(eat, w1e, b1r, w2b, b2r)

    ei_flat = edge_index.reshape(-1)
    cnts2 = _scatter_cnts(ei_flat)
    sums2 = _scatter_sums(eo, ei_flat)

    edge_final = pl.pallas_call(
        _edge_ef_kernel,
        grid=(E // BE,),
        in_specs=[
            eat_spec,
            full(w1e.shape), full(b1r.shape),
            full(W2.shape), full(b2r.shape),
            full(We1.shape), full(be1r.shape),
            full(We2.shape), full(be2r.shape),
            full(We3.shape), full(be3r.shape),
        ],
        out_specs=pl.BlockSpec((BE, GC), lambda i: (i, 0)),
        out_shape=jax.ShapeDtypeStruct((E, GC), jnp.float32),
    )(eat, w1e, b1r, w2b, b2r, we1b, be1r, we2b, be2r, we3b, be3r)

    node_final = pl.pallas_call(
        _node_block_kernel,
        grid=(N // BN,),
        in_specs=[
            pl.BlockSpec((NCORE, BN, GC), lambda i: (0, i, 0)),
            pl.BlockSpec((NCORE, BN, GC), lambda i: (0, i, 0)),
            full(Wn1.shape), full(bn1r.shape),
            full(Wn2.shape), full(bn2r.shape),
            full(Wn3.shape), full(bn3r.shape),
        ],
        out_specs=pl.BlockSpec((BN, GC), lambda i: (i, 0)),
        out_shape=jax.ShapeDtypeStruct((N, GC), jnp.float32),
    )(sums2, cnts2, Wn1, bn1r, Wn2, bn2r, Wn3, bn3r)

    return node_final, edge_final
